# baseline (speedup 1.0000x reference)
import functools

import jax
import jax.numpy as jnp
from jax import lax
from jax.experimental import pallas as pl
from jax.experimental.pallas import tpu as pltpu
from jax.experimental.pallas import tpu_sc as plsc

NUM_EMB = 100000
DIM = 64
NB = 4096
NSQ = 50

_info = plsc.get_sparse_core_info()
NC = _info.num_cores
NSUB = _info.num_subcores
NW = NC * NSUB
BPW = NB // NW

_mesh = plsc.VectorSubcoreMesh(core_axis_name="c", subcore_axis_name="s")


@functools.partial(
    pl.kernel,
    mesh=_mesh,
    out_type=jax.ShapeDtypeStruct((NSQ * (DIM // 8) * NW * 8 * BPW,), jnp.float32),
    compiler_params=pltpu.CompilerParams(
        use_tc_tiling_on_sc=False, needs_layout_passes=False
    ),
    scratch_types=[
        pltpu.VMEM((NSQ, BPW), jnp.int32),
        pltpu.VMEM((2, BPW, DIM), jnp.float32),
        pltpu.VMEM((2, DIM * BPW), jnp.float32),
        pltpu.SemaphoreType.DMA,
        pltpu.SemaphoreType.DMA,
        pltpu.SemaphoreType.DMA,
        pltpu.SemaphoreType.DMA,
    ],
)
def _gather_kernel(table_hbm, xt_hbm, out2, idx_c, gbuf, tbuf, g0, g1, o0, o1):
    gsem = (g0, g1)
    osem = (o0, o1)
    wid = lax.axis_index("s") * NC + lax.axis_index("c")
    b0 = wid * BPW
    pltpu.sync_copy(xt_hbm.at[:, pl.ds(b0, BPW)], idx_c)

    iota = lax.iota(jnp.int32, 16)
    wrap = [lax.rem(iota + j, 16) for j in range(16)]
    ldv = [iota * DIM + w for w in wrap]
    stv = [w * BPW + iota for w in wrap]
    LDL = (16 - 1) * DIM + 16
    STL = (16 - 1) * BPW + 16

    def gather(s, p):
        return pltpu.async_copy(table_hbm.at[idx_c.at[s]], gbuf.at[p], gsem[p])

    def put_word0(s, dh):
        return (((s * (DIM // 8) + dh) * NW + wid) * 8) * BPW

    def puts(s, p):
        for dh in range(DIM // 8):
            pltpu.async_copy(
                tbuf.at[p, pl.ds(dh * 8 * BPW, 8 * BPW)],
                out2.at[pl.ds(put_word0(s, dh), 8 * BPW)],
                osem[p],
            )

    def wait_puts(s, p):
        for dh in range(DIM // 8):
            pltpu.make_async_copy(
                tbuf.at[p, pl.ds(dh * 8 * BPW, 8 * BPW)],
                out2.at[pl.ds(put_word0(s, dh), 8 * BPW)],
                osem[p],
            ).wait()

    def transpose(p):
        gf = gbuf.at[p]
        tf = tbuf.at[p]

        def tb(k, carry):
            for u in range(4):
                i8 = k * 4 + u
                rowv = i8 * 16 + iota
                for d0 in range(0, DIM, 16):
                    sts = tf.at[pl.ds(d0 * BPW + i8 * 16, STL)]
                    for j in range(16):
                        v = plsc.load_gather(gf, [rowv, d0 + wrap[j]])
                        plsc.store_scatter(sts, [stv[j]], v)
            return carry

        lax.fori_loop(0, BPW // 64, tb, 0)

    gather(0, 0)
    gather(1, 1)

    def body(k, carry):
        se = 2 * k
        so = se + 1

        pltpu.make_async_copy(table_hbm.at[idx_c.at[se]], gbuf.at[0], g0).wait()

        @pl.when(k > 0)
        def _drain_even():
            wait_puts(se - 2, 0)

        transpose(0)
        puts(se, 0)

        @pl.when(k < NSQ // 2 - 1)
        def _next_even():
            gather(se + 2, 0)

        pltpu.make_async_copy(table_hbm.at[idx_c.at[so]], gbuf.at[1], g1).wait()

        @pl.when(k > 0)
        def _drain_odd():
            wait_puts(so - 2, 1)

        transpose(1)
        puts(so, 1)

        @pl.when(k < NSQ // 2 - 1)
        def _next_odd():
            gather(so + 2, 1)

        return carry

    lax.fori_loop(0, NSQ // 2, body, 0)
    wait_puts(NSQ - 2, 0)
    wait_puts(NSQ - 1, 1)


def kernel(x, table):
    xt = x.T.astype(jnp.int32)
    out1 = _gather_kernel(table, xt)
    out5 = out1.reshape(NSQ, DIM // 8, NW, 8, BPW)
    return out5.transpose(2, 4, 0, 1, 3).reshape(NB, NSQ, DIM)


# speedup vs baseline: 1.1687x; 1.1687x over previous
import functools

import jax
import jax.numpy as jnp
from jax import lax
from jax.experimental import pallas as pl
from jax.experimental.pallas import tpu as pltpu
from jax.experimental.pallas import tpu_sc as plsc

NUM_EMB = 100000
DIM = 64
NB = 4096
NSQ = 50

_info = plsc.get_sparse_core_info()
NC = _info.num_cores
NSUB = _info.num_subcores
NW = NC * NSUB
BPW = NB // NW

_mesh = plsc.VectorSubcoreMesh(core_axis_name="c", subcore_axis_name="s")


@functools.partial(
    pl.kernel,
    mesh=_mesh,
    out_type=jax.ShapeDtypeStruct((NSQ * (DIM // 8) * NW * 8 * BPW,), jnp.float32),
    compiler_params=pltpu.CompilerParams(
        use_tc_tiling_on_sc=False, needs_layout_passes=False
    ),
    scratch_types=[
        pltpu.VMEM((NSQ, BPW), jnp.int32),
        pltpu.VMEM((2, BPW, DIM), jnp.float32),
        pltpu.VMEM((2, DIM * BPW), jnp.float32),
        pltpu.SemaphoreType.DMA,
        pltpu.SemaphoreType.DMA,
        pltpu.SemaphoreType.DMA,
        pltpu.SemaphoreType.DMA,
    ],
)
def _gather_kernel(table_hbm, xt_hbm, out2, idx_c, gbuf, tbuf, g0, g1, o0, o1):
    gsem = (g0, g1)
    osem = (o0, o1)
    wid = lax.axis_index("s") * NC + lax.axis_index("c")
    b0 = wid * BPW
    pltpu.sync_copy(xt_hbm.at[:, pl.ds(b0, BPW)], idx_c)

    iota = lax.iota(jnp.int32, 16)
    wrap = [lax.rem(iota + j, 16) for j in range(16)]
    ldv = [iota * DIM + w for w in wrap]
    stv = [w * BPW + iota for w in wrap]
    LDL = (16 - 1) * DIM + 16
    STL = (16 - 1) * BPW + 16

    def gather(s, p):
        return pltpu.async_copy(table_hbm.at[idx_c.at[s]], gbuf.at[p], gsem[p])

    def put_word0(s, dh):
        return (((s * (DIM // 8) + dh) * NW + wid) * 8) * BPW

    def puts(s, p):
        for dh in range(DIM // 8):
            pltpu.async_copy(
                tbuf.at[p, pl.ds(dh * 8 * BPW, 8 * BPW)],
                out2.at[pl.ds(put_word0(s, dh), 8 * BPW)],
                osem[p],
            )

    def wait_puts(s, p):
        for dh in range(DIM // 8):
            pltpu.make_async_copy(
                tbuf.at[p, pl.ds(dh * 8 * BPW, 8 * BPW)],
                out2.at[pl.ds(put_word0(s, dh), 8 * BPW)],
                osem[p],
            ).wait()

    def transpose(p):
        gf = gbuf.at[p]
        tf = tbuf.at[p]

        def tb(k, carry):
            for u in range(2):
                i8 = k * 2 + u
                rowv = i8 * 16 + iota
                for d0 in range(0, DIM, 16):
                    sts = tf.at[pl.ds(d0 * BPW + i8 * 16, STL)]
                    for j in range(16):
                        v = plsc.load_gather(gf, [rowv, d0 + wrap[j]])
                        plsc.store_scatter(sts, [stv[j]], v)
            return carry

        lax.fori_loop(0, BPW // 32, tb, 0)

    gather(0, 0)
    gather(1, 1)

    def body(k, carry):
        se = 2 * k
        so = se + 1

        pltpu.make_async_copy(table_hbm.at[idx_c.at[se]], gbuf.at[0], g0).wait()

        @pl.when(k > 0)
        def _drain_even():
            wait_puts(se - 2, 0)

        transpose(0)
        puts(se, 0)

        @pl.when(k < NSQ // 2 - 1)
        def _next_even():
            gather(se + 2, 0)

        pltpu.make_async_copy(table_hbm.at[idx_c.at[so]], gbuf.at[1], g1).wait()

        @pl.when(k > 0)
        def _drain_odd():
            wait_puts(so - 2, 1)

        transpose(1)
        puts(so, 1)

        @pl.when(k < NSQ // 2 - 1)
        def _next_odd():
            gather(so + 2, 1)

        return carry

    lax.fori_loop(0, NSQ // 2, body, 0)
    wait_puts(NSQ - 2, 0)
    wait_puts(NSQ - 1, 1)


def kernel(x, table):
    xt = x.T.astype(jnp.int32)
    out1 = _gather_kernel(table, xt)
    out5 = out1.reshape(NSQ, DIM // 8, NW, 8, BPW)
    return out5.transpose(2, 4, 0, 1, 3).reshape(NB, NSQ, DIM)
